# baseline (speedup 1.0000x reference)
import functools

import jax
import jax.numpy as jnp
from jax import lax
from jax.experimental import pallas as pl
from jax.experimental.pallas import tpu as pltpu
from jax.experimental.pallas import tpu_sc as plsc

N = 10000
E = 320000
D = 128

NC = 2
NS = 16
NW = NC * NS
EDGES_PER_TILE = E // NW
CHUNK = 80
CHUNKS_PER_TILE = EDGES_PER_TILE // CHUNK
STRIPE = 640


def _seg_body(g_hbm, packed_hbm, zero_hbm, out_hbm,
              acc, packed_v, rows0, rows1, src0, src1, dst0, dst1,
              gsem0, gsem1, ssem0, ssem1):
    c = lax.axis_index("c")
    s = lax.axis_index("s")
    wid = s * NC + c
    row_lo = pl.multiple_of(jnp.where(s == NS - 1, N - STRIPE, s * STRIPE), 8)
    pltpu.sync_copy(zero_hbm.at[pl.ds(row_lo, STRIPE)],
                    acc.at[pl.ds(row_lo, STRIPE)])
    pltpu.sync_copy(packed_hbm.at[wid], packed_v.at[pl.ds(0, CHUNKS_PER_TILE)])
    plsc.subcore_barrier()

    def unpack(j, src_b, dst_b):
        for k in range(CHUNK // 16):
            v = packed_v[j, pl.ds(k * 16, 16)]
            src_b[pl.ds(k * 16, 16)] = jnp.bitwise_and(v, 0xFFFF)
            dst_b[pl.ds(k * 16, 16)] = lax.shift_right_logical(v, 16)

    def wait_g(sem, buf):
        pltpu.make_async_copy(g_hbm.at[src0], buf, sem).wait()

    def wait_s(sem, buf):
        pltpu.make_async_copy(buf, acc.at[src0], sem).wait()

    unpack(0, src0, dst0)
    unpack(1, src1, dst1)
    pltpu.async_copy(g_hbm.at[src0], rows0, gsem0)
    pltpu.async_copy(g_hbm.at[src1], rows1, gsem1)

    @pl.loop(0, CHUNKS_PER_TILE - 3, step=2)
    def _(j):
        wait_g(gsem0, rows0)
        pltpu.async_copy(rows0, acc.at[dst0], ssem0, add=True)
        wait_g(gsem1, rows1)
        pltpu.async_copy(rows1, acc.at[dst1], ssem1, add=True)
        wait_s(ssem0, rows0)
        unpack(j + 2, src0, dst0)
        pltpu.async_copy(g_hbm.at[src0], rows0, gsem0)
        wait_s(ssem1, rows1)
        unpack(j + 3, src1, dst1)
        pltpu.async_copy(g_hbm.at[src1], rows1, gsem1)

    wait_g(gsem0, rows0)
    pltpu.async_copy(rows0, acc.at[dst0], ssem0, add=True)
    wait_g(gsem1, rows1)
    pltpu.async_copy(rows1, acc.at[dst1], ssem1, add=True)
    wait_s(ssem0, rows0)
    unpack(CHUNKS_PER_TILE - 1, src0, dst0)
    pltpu.async_copy(g_hbm.at[src0], rows0, gsem0)
    wait_g(gsem0, rows0)
    pltpu.sync_copy(rows0, acc.at[dst0], add=True)
    wait_s(ssem1, rows1)

    plsc.subcore_barrier()
    pltpu.sync_copy(acc.at[pl.ds(row_lo, STRIPE)],
                    out_hbm.at[c].at[pl.ds(row_lo, STRIPE)])


_seg_partial = functools.partial(
    pl.kernel,
    out_type=jax.ShapeDtypeStruct((NC, N, D), jnp.float32),
    mesh=plsc.VectorSubcoreMesh(core_axis_name="c", subcore_axis_name="s"),
    scratch_types=[
        pltpu.VMEM_SHARED((N, D), jnp.float32),
        pltpu.VMEM((128, CHUNK), jnp.int32),
        pltpu.VMEM((CHUNK, D), jnp.float32),
        pltpu.VMEM((CHUNK, D), jnp.float32),
        pltpu.VMEM((CHUNK,), jnp.int32),
        pltpu.VMEM((CHUNK,), jnp.int32),
        pltpu.VMEM((CHUNK,), jnp.int32),
        pltpu.VMEM((CHUNK,), jnp.int32),
        pltpu.SemaphoreType.DMA,
        pltpu.SemaphoreType.DMA,
        pltpu.SemaphoreType.DMA,
        pltpu.SemaphoreType.DMA,
    ],
)(_seg_body)


def _tc_first_body(x_ref, wr_ref, wo_ref, b_ref, g_ref, r_ref):
    xv = x_ref[...]
    g_ref[...] = jnp.dot(xv, wr_ref[...], preferred_element_type=jnp.float32)
    r_ref[...] = jnp.dot(xv, wo_ref[...], preferred_element_type=jnp.float32) + b_ref[...]


def _tc_mid_body(p_ref, r_ref, wr_ref, wo_ref, b_ref, g_ref, ro_ref):
    h = jnp.maximum(p_ref[0] + p_ref[1] + r_ref[...], 0.0)
    g_ref[...] = jnp.dot(h, wr_ref[...], preferred_element_type=jnp.float32)
    ro_ref[...] = jnp.dot(h, wo_ref[...], preferred_element_type=jnp.float32) + b_ref[...]


def _tc_last_body(p_ref, r_ref, o_ref):
    o_ref[...] = p_ref[0] + p_ref[1] + r_ref[...]


_nd = jax.ShapeDtypeStruct((N, D), jnp.float32)

_tc_first = pl.pallas_call(_tc_first_body, out_shape=(_nd, _nd))
_tc_mid = pl.pallas_call(_tc_mid_body, out_shape=(_nd, _nd))
_tc_last = pl.pallas_call(_tc_last_body, out_shape=_nd)


def kernel(x, edge_index, W_rel0, b_rel0, W_root0, W_rel1, b_rel1, W_root1,
           W_rel2, b_rel2, W_root2):
    packed = ((edge_index[1] << 16) | edge_index[0]).reshape(
        NW, CHUNKS_PER_TILE, CHUNK)
    zeros = jnp.zeros((N, D), jnp.float32)
    params = [(W_rel0, b_rel0, W_root0), (W_rel1, b_rel1, W_root1),
              (W_rel2, b_rel2, W_root2)]

    g, r = _tc_first(x, W_rel0.T, W_root0.T, b_rel0.reshape(1, D))
    for i in (1, 2):
        p = _seg_partial(g, packed, zeros)
        Wr, br, Wo = params[i]
        g, r = _tc_mid(p, r, Wr.T, Wo.T, br.reshape(1, D))
    p = _seg_partial(g, packed, zeros)
    return _tc_last(p, r)


# speedup vs baseline: 1.2070x; 1.2070x over previous
import functools

import jax
import jax.numpy as jnp
from jax import lax
from jax.experimental import pallas as pl
from jax.experimental.pallas import tpu as pltpu
from jax.experimental.pallas import tpu_sc as plsc

N = 10000
E = 320000
D = 128

NC = 2
NS = 16
NW = NC * NS
EDGES_PER_TILE = E // NW
CHUNK = 80
CHUNKS_PER_TILE = EDGES_PER_TILE // CHUNK
STRIPE = 640


def _seg_body(g_hbm, packed_hbm, zero_hbm, out_hbm,
              acc, packed_v, rows0, rows1, src0, src1, dst0, dst1,
              gsem0, gsem1, ssem0):
    c = lax.axis_index("c")
    s = lax.axis_index("s")
    wid = s * NC + c
    row_lo = pl.multiple_of(jnp.where(s == NS - 1, N - STRIPE, s * STRIPE), 8)
    pltpu.async_copy(zero_hbm.at[pl.ds(row_lo, STRIPE)],
                     acc.at[pl.ds(row_lo, STRIPE)], ssem0)
    pltpu.sync_copy(packed_hbm.at[wid], packed_v.at[pl.ds(0, CHUNKS_PER_TILE)])
    pltpu.make_async_copy(zero_hbm.at[pl.ds(row_lo, STRIPE)],
                          acc.at[pl.ds(row_lo, STRIPE)], ssem0).wait()
    plsc.subcore_barrier()

    def unpack(j, src_b, dst_b):
        for k in range(CHUNK // 16):
            v = packed_v[j, pl.ds(k * 16, 16)]
            src_b[pl.ds(k * 16, 16)] = jnp.bitwise_and(v, 0xFFFF)
            dst_b[pl.ds(k * 16, 16)] = lax.shift_right_logical(v, 16)

    last = CHUNKS_PER_TILE - 1
    unpack(0, src0, dst0)
    unpack(1, src1, dst1)
    pltpu.async_copy(g_hbm.at[src0], rows0, gsem0)

    @pl.loop(0, last, step=2)
    def _(j):
        pltpu.make_async_copy(g_hbm.at[src0], rows0, gsem0).wait()
        pltpu.async_copy(g_hbm.at[src1], rows1, gsem1)
        pltpu.sync_copy(rows0, acc.at[dst0], add=True)
        unpack(j + 2, src0, dst0)
        pltpu.async_copy(g_hbm.at[src0], rows0, gsem0)
        pltpu.make_async_copy(g_hbm.at[src1], rows1, gsem1).wait()
        pltpu.sync_copy(rows1, acc.at[dst1], add=True)
        unpack(j + 3, src1, dst1)

    pltpu.make_async_copy(g_hbm.at[src0], rows0, gsem0).wait()
    pltpu.sync_copy(rows0, acc.at[dst0], add=True)

    plsc.subcore_barrier()
    pltpu.sync_copy(acc.at[pl.ds(row_lo, STRIPE)],
                    out_hbm.at[c].at[pl.ds(row_lo, STRIPE)])


_seg_partial = functools.partial(
    pl.kernel,
    out_type=jax.ShapeDtypeStruct((NC, N, D), jnp.float32),
    mesh=plsc.VectorSubcoreMesh(core_axis_name="c", subcore_axis_name="s"),
    scratch_types=[
        pltpu.VMEM_SHARED((N, D), jnp.float32),
        pltpu.VMEM((128, CHUNK), jnp.int32),
        pltpu.VMEM((CHUNK, D), jnp.float32),
        pltpu.VMEM((CHUNK, D), jnp.float32),
        pltpu.VMEM((CHUNK,), jnp.int32),
        pltpu.VMEM((CHUNK,), jnp.int32),
        pltpu.VMEM((CHUNK,), jnp.int32),
        pltpu.VMEM((CHUNK,), jnp.int32),
        pltpu.SemaphoreType.DMA,
        pltpu.SemaphoreType.DMA,
        pltpu.SemaphoreType.DMA,
    ],
)(_seg_body)


def _tc_first_body(x_ref, wr_ref, wo_ref, b_ref, g_ref, r_ref):
    xv = x_ref[...]
    g_ref[...] = jnp.dot(xv, wr_ref[...], preferred_element_type=jnp.float32)
    r_ref[...] = jnp.dot(xv, wo_ref[...], preferred_element_type=jnp.float32) + b_ref[...]


def _tc_mid_body(p_ref, r_ref, wr_ref, wo_ref, b_ref, g_ref, ro_ref):
    h = jnp.maximum(p_ref[0] + p_ref[1] + r_ref[...], 0.0)
    g_ref[...] = jnp.dot(h, wr_ref[...], preferred_element_type=jnp.float32)
    ro_ref[...] = jnp.dot(h, wo_ref[...], preferred_element_type=jnp.float32) + b_ref[...]


def _tc_last_body(p_ref, r_ref, o_ref):
    o_ref[...] = p_ref[0] + p_ref[1] + r_ref[...]


_nd = jax.ShapeDtypeStruct((N, D), jnp.float32)
_TB = 1000
_row_spec = pl.BlockSpec((_TB, D), lambda i: (i, 0))
_p_spec = pl.BlockSpec((NC, _TB, D), lambda i: (0, i, 0))
_w_spec = pl.BlockSpec((D, D), lambda i: (0, 0))
_b_spec = pl.BlockSpec((1, D), lambda i: (0, 0))

_tc_first = pl.pallas_call(
    _tc_first_body, grid=(N // _TB,),
    in_specs=[_row_spec, _w_spec, _w_spec, _b_spec],
    out_specs=(_row_spec, _row_spec), out_shape=(_nd, _nd))
_tc_mid = pl.pallas_call(
    _tc_mid_body, grid=(N // _TB,),
    in_specs=[_p_spec, _row_spec, _w_spec, _w_spec, _b_spec],
    out_specs=(_row_spec, _row_spec), out_shape=(_nd, _nd))
_tc_last = pl.pallas_call(
    _tc_last_body, grid=(N // _TB,),
    in_specs=[_p_spec, _row_spec], out_specs=_row_spec, out_shape=_nd)


def kernel(x, edge_index, W_rel0, b_rel0, W_root0, W_rel1, b_rel1, W_root1,
           W_rel2, b_rel2, W_root2):
    packed = ((edge_index[1] << 16) | edge_index[0]).reshape(
        NW, CHUNKS_PER_TILE, CHUNK)
    zeros = jnp.zeros((N, D), jnp.float32)
    params = [(W_rel0, b_rel0, W_root0), (W_rel1, b_rel1, W_root1),
              (W_rel2, b_rel2, W_root2)]

    g, r = _tc_first(x, W_rel0.T, W_root0.T, b_rel0.reshape(1, D))
    for i in (1, 2):
        p = _seg_partial(g, packed, zeros)
        Wr, br, Wo = params[i]
        g, r = _tc_mid(p, r, Wr.T, Wo.T, br.reshape(1, D))
    p = _seg_partial(g, packed, zeros)
    return _tc_last(p, r)


# speedup vs baseline: 1.2398x; 1.0271x over previous
import functools

import jax
import jax.numpy as jnp
from jax import lax
from jax.experimental import pallas as pl
from jax.experimental.pallas import tpu as pltpu
from jax.experimental.pallas import tpu_sc as plsc

N = 10000
E = 320000
D = 128

NC = 2
NS = 16
NW = NC * NS
EDGES_PER_TILE = E // NW
CHUNK = 80
CHUNKS_PER_TILE = EDGES_PER_TILE // CHUNK
STRIPE = 640


def _seg_body(g_hbm, packed_hbm, zero_hbm, out_hbm,
              acc, packed_v, rows0, rows1, src0, src1, dst0, dst1,
              gsem0, gsem1, ssem0):
    c = lax.axis_index("c")
    s = lax.axis_index("s")
    wid = s * NC + c
    row_lo = pl.multiple_of(jnp.where(s == NS - 1, N - STRIPE, s * STRIPE), 8)
    pltpu.async_copy(zero_hbm.at[pl.ds(row_lo, STRIPE)],
                     acc.at[pl.ds(row_lo, STRIPE)], ssem0)
    pltpu.sync_copy(packed_hbm.at[wid], packed_v.at[pl.ds(0, CHUNKS_PER_TILE)])
    pltpu.make_async_copy(zero_hbm.at[pl.ds(row_lo, STRIPE)],
                          acc.at[pl.ds(row_lo, STRIPE)], ssem0).wait()
    plsc.subcore_barrier()

    def unpack(j, src_b, dst_b):
        for k in range(CHUNK // 16):
            v = packed_v[j, pl.ds(k * 16, 16)]
            src_b[pl.ds(k * 16, 16)] = jnp.bitwise_and(v, 0xFFFF)
            dst_b[pl.ds(k * 16, 16)] = lax.shift_right_logical(v, 16)

    last = CHUNKS_PER_TILE - 1
    unpack(0, src0, dst0)
    unpack(1, src1, dst1)
    pltpu.async_copy(g_hbm.at[src0], rows0, gsem0)

    @pl.loop(0, last, step=2)
    def _(j):
        pltpu.make_async_copy(g_hbm.at[src0], rows0, gsem0).wait()
        pltpu.async_copy(g_hbm.at[src1], rows1, gsem1)
        pltpu.sync_copy(rows0, acc.at[dst0], add=True)
        unpack(j + 2, src0, dst0)
        pltpu.async_copy(g_hbm.at[src0], rows0, gsem0)
        pltpu.make_async_copy(g_hbm.at[src1], rows1, gsem1).wait()
        pltpu.sync_copy(rows1, acc.at[dst1], add=True)
        unpack(j + 3, src1, dst1)

    pltpu.make_async_copy(g_hbm.at[src0], rows0, gsem0).wait()
    pltpu.sync_copy(rows0, acc.at[dst0], add=True)

    plsc.subcore_barrier()
    pltpu.sync_copy(acc.at[pl.ds(row_lo, STRIPE)],
                    out_hbm.at[c].at[pl.ds(row_lo, STRIPE)])


_seg_partial = functools.partial(
    pl.kernel,
    out_type=jax.ShapeDtypeStruct((NC, N, D), jnp.float32),
    mesh=plsc.VectorSubcoreMesh(core_axis_name="c", subcore_axis_name="s"),
    scratch_types=[
        pltpu.VMEM_SHARED((N, D), jnp.float32),
        pltpu.VMEM((128, CHUNK), jnp.int32),
        pltpu.VMEM((CHUNK, D), jnp.float32),
        pltpu.VMEM((CHUNK, D), jnp.float32),
        pltpu.VMEM((CHUNK,), jnp.int32),
        pltpu.VMEM((CHUNK,), jnp.int32),
        pltpu.VMEM((CHUNK,), jnp.int32),
        pltpu.VMEM((CHUNK,), jnp.int32),
        pltpu.SemaphoreType.DMA,
        pltpu.SemaphoreType.DMA,
        pltpu.SemaphoreType.DMA,
    ],
)(_seg_body)


def _tc_first_body(x_ref, wr_ref, wo_ref, b_ref, g_ref, r_ref):
    xv = x_ref[...]
    g_ref[...] = jnp.dot(xv, wr_ref[...], preferred_element_type=jnp.float32)
    r_ref[...] = jnp.dot(xv, wo_ref[...], preferred_element_type=jnp.float32) + b_ref[...]


def _tc_mid_body(p_ref, r_ref, wr_ref, wo_ref, b_ref, g_ref, ro_ref):
    h = jnp.maximum(p_ref[0] + p_ref[1] + r_ref[...], 0.0)
    g_ref[...] = jnp.dot(h, wr_ref[...], preferred_element_type=jnp.float32)
    ro_ref[...] = jnp.dot(h, wo_ref[...], preferred_element_type=jnp.float32) + b_ref[...]


def _tc_last_body(p_ref, r_ref, o_ref):
    o_ref[...] = p_ref[0] + p_ref[1] + r_ref[...]


_nd = jax.ShapeDtypeStruct((N, D), jnp.float32)

_tc_first = pl.pallas_call(_tc_first_body, out_shape=(_nd, _nd))
_tc_mid = pl.pallas_call(_tc_mid_body, out_shape=(_nd, _nd))
_tc_last = pl.pallas_call(_tc_last_body, out_shape=_nd)


def kernel(x, edge_index, W_rel0, b_rel0, W_root0, W_rel1, b_rel1, W_root1,
           W_rel2, b_rel2, W_root2):
    packed = ((edge_index[1] << 16) | edge_index[0]).reshape(
        NW, CHUNKS_PER_TILE, CHUNK)
    zeros = jnp.zeros((N, D), jnp.float32)
    params = [(W_rel0, b_rel0, W_root0), (W_rel1, b_rel1, W_root1),
              (W_rel2, b_rel2, W_root2)]

    g, r = _tc_first(x, W_rel0.T, W_root0.T, b_rel0.reshape(1, D))
    for i in (1, 2):
        p = _seg_partial(g, packed, zeros)
        Wr, br, Wo = params[i]
        g, r = _tc_mid(p, r, Wr.T, Wo.T, br.reshape(1, D))
    p = _seg_partial(g, packed, zeros)
    return _tc_last(p, r)
